# baseline (speedup 1.0000x reference)
import jax
import jax.numpy as jnp
from jax.experimental import pallas as pl

DIM = 96
HEADS = 8
DH = DIM // HEADS
KV = 2 * DH
T = 197
G = HEADS * T


def _prep_body(x_ref, wq_ref, bq_ref, wk_ref, bk_ref, wv_ref, bv_ref,
               Wq_ref, Wk_ref, Wv_ref, q_ref, k_ref, v_ref):
    xv = x_ref[0]
    cls = xv[0:1, :]
    xs = xv[1:, :]
    xsr = xs.reshape(14, 14, 96)
    zr = jnp.zeros((1, 14, 96), jnp.float32)
    rows16 = jnp.concatenate([zr, xsr, zr], axis=0)
    zc = jnp.zeros((16, 1, 96), jnp.float32)
    p = jnp.concatenate([zc, rows16, zc], axis=1)

    def branch(w_ref, b_ref, W_ref, out_ref):
        acc = jnp.zeros((14, 14, 96), jnp.float32)
        for dy in range(3):
            for dx in range(3):
                tap = p[dy:dy + 14, dx:dx + 14, :]
                acc = acc + tap * w_ref[dy * 3 + dx][None, None, :]
        y = acc + b_ref[0][None, None, :]
        full = jnp.concatenate([cls, y.reshape(196, 96)], axis=0)
        out_ref[0] = jnp.dot(full, W_ref[:], preferred_element_type=jnp.float32)

    branch(wq_ref, bq_ref, Wq_ref, q_ref)
    branch(wk_ref, bk_ref, Wk_ref, k_ref)
    branch(wv_ref, bv_ref, Wv_ref, v_ref)


def _main_body(asg_ref, asgA_ref, kvg_ref, qp_ref, kp_ref, vp_ref, first_ref,
               rep_ref, wc_ref, b2_ref, lbig_ref, mbig_ref, rsum_ref, out_ref):
    a = asg_ref[0]
    rem2 = a[:, 1:] * 2.0
    mp = jnp.max(rem2, axis=-1, keepdims=True)
    sp = jnp.sum(jnp.exp(rem2 - mp), axis=-1, keepdims=True)
    mn = jnp.max(-rem2, axis=-1, keepdims=True)
    sn = jnp.sum(jnp.exp(-rem2 - mn), axis=-1, keepdims=True)
    isp = 1.0 / sp
    isn = 1.0 / sn

    rawA = asgA_ref[0, 0]
    x2 = rawA * 2.0
    iota_p = jax.lax.broadcasted_iota(jnp.int32, (196, 8), 0)
    iota_j = jax.lax.broadcasted_iota(jnp.int32, (196, 8), 1)
    u8 = (iota_p * 8 + iota_j) // 196
    mpA = jnp.zeros((196, 8), jnp.float32)
    ispA = jnp.zeros((196, 8), jnp.float32)
    mnA = jnp.zeros((196, 8), jnp.float32)
    isnA = jnp.zeros((196, 8), jnp.float32)
    for u in range(8):
        msk = (u8 == u).astype(jnp.float32)
        mpA = mpA + msk * mp[u:u + 1, 0:1]
        ispA = ispA + msk * isp[u:u + 1, 0:1]
        mnA = mnA + msk * mn[u:u + 1, 0:1]
        isnA = isnA + msk * isn[u:u + 1, 0:1]
    posA = jnp.exp(x2 - mpA) * ispA
    negA = jnp.exp(-x2 - mnA) * isnA
    agsA = 0.7 * posA + 0.3 - 0.3 * negA

    A192 = jnp.dot(agsA, rep_ref[:], preferred_element_type=jnp.float32)
    kvg = kvg_ref[0]
    iota_pc = jax.lax.broadcasted_iota(jnp.int32, (196, 192), 0)
    iota_cc = jax.lax.broadcasted_iota(jnp.int32, (196, 192), 1)
    u192 = (iota_pc * 8 + iota_cc // 24) // 196
    K192 = jnp.zeros((196, 192), jnp.float32)
    for u in range(8):
        tile = jnp.concatenate([kvg[u:u + 1, :]] * 8, axis=1)
        K192 = K192 + (u192 == u).astype(jnp.float32) * tile
    f8 = A192 * K192

    f8s = f8.reshape(14, 7, 2, 192)
    jeven = f8s[:, :, 0, :]
    jodd = f8s[:, :, 1, :]
    zc1 = jnp.zeros((14, 1, 192), jnp.float32)
    ep = jnp.concatenate([zc1, jodd], axis=1)
    op = jnp.concatenate([jeven, zc1], axis=1)
    zr1 = jnp.zeros((1, 8, 192), jnp.float32)
    epp = jnp.concatenate([zr1, ep, zr1], axis=0).reshape(8, 2, 8, 192)
    opp = jnp.concatenate([zr1, op, zr1], axis=0).reshape(8, 2, 8, 192)

    acc = jnp.zeros((49, 192), jnp.float32)
    for dy in range(3):
        a0 = 1 if dy == 2 else 0
        e0 = dy % 2
        for dx in range(3):
            b0 = 1 if dx == 2 else 0
            imgr = opp if dx == 1 else epp
            tap = imgr[a0:a0 + 7, e0, b0:b0 + 7, :].reshape(49, 192)
            acc = acc + jnp.dot(tap, wc_ref[dy * 3 + dx],
                                preferred_element_type=jnp.float32)
    co = acc + b2_ref[0][None, :]

    ybig = jnp.dot(lbig_ref[:], co, preferred_element_type=jnp.float32)
    zbig = jnp.dot(ybig * mbig_ref[:], rsum_ref[:],
                   preferred_element_type=jnp.float32)
    z3 = zbig.reshape(24, 49, 8)
    kparts = z3[:12]
    vparts = z3[12:]
    qsT = qp_ref[0, 0] * (96.0 ** -0.5)
    logitsT = jnp.sum(kparts * qsT[:, None, :], axis=0)
    fvT = first_ref[0, 0]
    kprT = kp_ref[0, 0]
    vprT = vp_ref[0, 0]
    logit0T = jnp.sum(qsT * kprT, axis=0, keepdims=True) * fvT
    v0T = vprT * fvT
    m = jnp.maximum(jnp.max(logitsT, axis=0, keepdims=True), logit0T)
    e = jnp.exp(logitsT - m)
    e0 = jnp.exp(logit0T - m)
    den = jnp.sum(e, axis=0, keepdims=True) + e0
    outT = (jnp.sum(vparts * e[None, :, :], axis=1) + e0 * v0T) / den
    out_ref[0, 0] = outT


def _proj_body(x_ref, Wo_ref, bo_ref, out_ref):
    out_ref[...] = jnp.dot(x_ref[...], Wo_ref[...],
                           preferred_element_type=jnp.float32) + bo_ref[0][None, :]


@jax.jit
def _run(x, asg, wq_t, bq, wk_t, bk, wv_t, bv, Wq, Wk, Wv, Wc_t, b2, Wo, bo):
    B = x.shape[0]
    prep = pl.pallas_call(
        _prep_body,
        grid=(B,),
        in_specs=[
            pl.BlockSpec((1, T, DIM), lambda b: (b, 0, 0)),
            pl.BlockSpec((9, DIM), lambda b: (0, 0)),
            pl.BlockSpec((1, DIM), lambda b: (0, 0)),
            pl.BlockSpec((9, DIM), lambda b: (0, 0)),
            pl.BlockSpec((1, DIM), lambda b: (0, 0)),
            pl.BlockSpec((9, DIM), lambda b: (0, 0)),
            pl.BlockSpec((1, DIM), lambda b: (0, 0)),
            pl.BlockSpec((DIM, DIM), lambda b: (0, 0)),
            pl.BlockSpec((DIM, DIM), lambda b: (0, 0)),
            pl.BlockSpec((DIM, DIM), lambda b: (0, 0)),
        ],
        out_specs=[
            pl.BlockSpec((1, T, DIM), lambda b: (b, 0, 0)),
            pl.BlockSpec((1, T, DIM), lambda b: (b, 0, 0)),
            pl.BlockSpec((1, T, DIM), lambda b: (b, 0, 0)),
        ],
        out_shape=[jax.ShapeDtypeStruct((B, T, DIM), jnp.float32)] * 3,
    )
    qproj, kproj, vproj = prep(x, wq_t, bq, wk_t, bk, wv_t, bv, Wq, Wk, Wv)

    kh = kproj.reshape(B, T, HEADS, DH).transpose(0, 2, 1, 3)
    vh = vproj.reshape(B, T, HEADS, DH).transpose(0, 2, 1, 3)
    kv2g = jnp.concatenate([kh, vh], axis=-1).reshape(B, G, KV)
    asg2 = asg.reshape(B, G, T)
    asgA = asg2[:, :, 1:].reshape(B, T, 196, 8)
    first_arr = asg[:, :, :, 0].transpose(0, 2, 1).reshape(B, T, 1, HEADS)
    qp4 = qproj.reshape(B, T, HEADS, DH).transpose(0, 1, 3, 2)
    kp4 = kproj.reshape(B, T, HEADS, DH).transpose(0, 1, 3, 2)
    vp4 = vproj.reshape(B, T, HEADS, DH).transpose(0, 1, 3, 2)
    rep = jnp.repeat(jnp.eye(HEADS, dtype=jnp.float32), KV, axis=1)
    rows = jnp.arange(24 * 49)
    mm = 24 * (rows % 49) + rows // 49
    lbig = (jnp.arange(49)[None, :] == (mm % 49)[:, None]).astype(jnp.float32)
    mbig = ((jnp.arange(2 * DIM)[None, :] % KV) == (mm // 49)[:, None]).astype(jnp.float32)
    rsum = ((jnp.arange(2 * DIM)[:, None] // KV) == jnp.arange(HEADS)[None, :]).astype(jnp.float32)

    outT = pl.pallas_call(
        _main_body,
        grid=(B, T),
        in_specs=[
            pl.BlockSpec((1, 8, T), lambda b, t: (b, t, 0)),
            pl.BlockSpec((1, 1, 196, 8), lambda b, t: (b, t, 0, 0)),
            pl.BlockSpec((1, 8, KV), lambda b, t: (b, t, 0)),
            pl.BlockSpec((1, 1, DH, HEADS), lambda b, t: (b, t, 0, 0)),
            pl.BlockSpec((1, 1, DH, HEADS), lambda b, t: (b, t, 0, 0)),
            pl.BlockSpec((1, 1, DH, HEADS), lambda b, t: (b, t, 0, 0)),
            pl.BlockSpec((1, 1, 1, HEADS), lambda b, t: (b, t, 0, 0)),
            pl.BlockSpec((HEADS, 2 * DIM), lambda b, t: (0, 0)),
            pl.BlockSpec((9, 2 * DIM, 2 * DIM), lambda b, t: (0, 0, 0)),
            pl.BlockSpec((1, 2 * DIM), lambda b, t: (0, 0)),
            pl.BlockSpec((24 * 49, 49), lambda b, t: (0, 0)),
            pl.BlockSpec((24 * 49, 2 * DIM), lambda b, t: (0, 0)),
            pl.BlockSpec((2 * DIM, HEADS), lambda b, t: (0, 0)),
        ],
        out_specs=pl.BlockSpec((1, 1, DH, HEADS), lambda b, t: (b, t, 0, 0)),
        out_shape=jax.ShapeDtypeStruct((B, T, DH, HEADS), jnp.float32),
    )(asg2, asgA, kv2g, qp4, kp4, vp4, first_arr, rep, Wc_t, b2,
      lbig, mbig, rsum)

    o96 = outT.transpose(0, 1, 3, 2).reshape(B * T, DIM)
    res = pl.pallas_call(
        _proj_body,
        grid=(1,),
        in_specs=[
            pl.BlockSpec((B * T, DIM), lambda i: (0, 0)),
            pl.BlockSpec((DIM, DIM), lambda i: (0, 0)),
            pl.BlockSpec((1, DIM), lambda i: (0, 0)),
        ],
        out_specs=pl.BlockSpec((B * T, DIM), lambda i: (0, 0)),
        out_shape=jax.ShapeDtypeStruct((B * T, DIM), jnp.float32),
    )(o96, Wo, bo)
    return res.reshape(B, T, DIM)


def kernel(x, h, w, attn_score_grad, conv_q_w, bn_q_g, bn_q_b, conv_k_w,
           bn_k_g, bn_k_b, conv_v_w, bn_v_g, bn_v_b, Wq, Wk, Wv, Cw, Cb,
           bn2_g, bn2_b, Wo, bo):
    eps = 1e-5
    sq = bn_q_g / jnp.sqrt(1.0 + eps)
    sk = bn_k_g / jnp.sqrt(1.0 + eps)
    sv = bn_v_g / jnp.sqrt(1.0 + eps)
    wq_t = (conv_q_w[:, 0] * sq[:, None, None]).transpose(1, 2, 0).reshape(9, DIM)
    wk_t = (conv_k_w[:, 0] * sk[:, None, None]).transpose(1, 2, 0).reshape(9, DIM)
    wv_t = (conv_v_w[:, 0] * sv[:, None, None]).transpose(1, 2, 0).reshape(9, DIM)
    s2 = bn2_g / jnp.sqrt(1.0 + eps)
    Wc_t = (Cw * s2[:, None, None, None]).transpose(2, 3, 1, 0).reshape(9, 2 * DIM, 2 * DIM)
    b2 = (Cb * s2 + bn2_b).reshape(1, 2 * DIM)
    return _run(x, attn_score_grad, wq_t, bn_q_b.reshape(1, DIM), wk_t,
                bn_k_b.reshape(1, DIM), wv_t, bn_v_b.reshape(1, DIM),
                Wq, Wk, Wv, Wc_t, b2, Wo, bo.reshape(1, DIM))


# speedup vs baseline: 2.0030x; 2.0030x over previous
import jax
import jax.numpy as jnp
from jax.experimental import pallas as pl

DIM = 96
HEADS = 8
DH = DIM // HEADS
KV = 2 * DH
T = 197
G = HEADS * T


def _prep_body(x_ref, wq_ref, bq_ref, wk_ref, bk_ref, wv_ref, bv_ref,
               Wq_ref, Wk_ref, Wv_ref, q_ref, k_ref, v_ref):
    xv = x_ref[0]
    cls = xv[0:1, :]
    xs = xv[1:, :]
    xsr = xs.reshape(14, 14, 96)
    zr = jnp.zeros((1, 14, 96), jnp.float32)
    rows16 = jnp.concatenate([zr, xsr, zr], axis=0)
    zc = jnp.zeros((16, 1, 96), jnp.float32)
    p = jnp.concatenate([zc, rows16, zc], axis=1)

    def branch(w_ref, b_ref, W_ref, out_ref):
        acc = jnp.zeros((14, 14, 96), jnp.float32)
        for dy in range(3):
            for dx in range(3):
                tap = p[dy:dy + 14, dx:dx + 14, :]
                acc = acc + tap * w_ref[dy * 3 + dx][None, None, :]
        y = acc + b_ref[0][None, None, :]
        full = jnp.concatenate([cls, y.reshape(196, 96)], axis=0)
        out_ref[0] = jnp.dot(full, W_ref[:], preferred_element_type=jnp.float32)

    branch(wq_ref, bq_ref, Wq_ref, q_ref)
    branch(wk_ref, bk_ref, Wk_ref, k_ref)
    branch(wv_ref, bv_ref, Wv_ref, v_ref)


NT = 4


def _main_body(asg_ref, asgA_ref, kvg_ref, qp_ref, kp_ref, vp_ref, first_ref,
               rep_ref, oh0_ref, ohd_ref, oh0T_ref, ohdT_ref, cm8T_ref,
               cmc_ref, sel_ref, wc_ref, b2_ref, gall_ref, mall_ref,
               rsumT_ref, out_ref):
    cm8T = cm8T_ref[:]
    cmc = cmc_ref[:]
    taps = [[] for _ in range(9)]
    for i in range(NT):
        rows = asg_ref[0, 8 * i:8 * i + 8, :]
        rem2 = rows[:, 1:] * 2.0
        mp = jnp.max(rem2, axis=-1, keepdims=True)
        sp = jnp.sum(jnp.exp(rem2 - mp), axis=-1, keepdims=True)
        mn = jnp.max(-rem2, axis=-1, keepdims=True)
        sn = jnp.sum(jnp.exp(-rem2 - mn), axis=-1, keepdims=True)
        s4 = jnp.concatenate([mp, 1.0 / sp, mn, 1.0 / sn], axis=1)
        s4T = s4.T
        c0 = jnp.dot(s4T, oh0T_ref[:], preferred_element_type=jnp.float32)
        cd = jnp.dot(s4T, ohdT_ref[:], preferred_element_type=jnp.float32)
        mpA = c0[0:1, :] + cm8T * cd[0:1, :]
        ispA = c0[1:2, :] + cm8T * cd[1:2, :]
        mnA = c0[2:3, :] + cm8T * cd[2:3, :]
        isnA = c0[3:4, :] + cm8T * cd[3:4, :]
        x2 = asgA_ref[0, i] * 2.0
        posA = jnp.exp(x2 - mpA) * ispA
        negA = jnp.exp(-x2 - mnA) * isnA
        agsAT = 0.7 * posA + 0.3 - 0.3 * negA

        A192 = jnp.dot(agsAT.T, rep_ref[:], preferred_element_type=jnp.float32)
        kvg = kvg_ref[0, 8 * i:8 * i + 8, :]
        tk = jnp.dot(kvg, sel_ref[:], preferred_element_type=jnp.float32)
        t0 = jnp.dot(oh0_ref[:], tk, preferred_element_type=jnp.float32)
        td = jnp.dot(ohd_ref[:], tk, preferred_element_type=jnp.float32)
        f8 = A192 * (t0 + cmc * td)

        f85 = f8.reshape(7, 2, 7, 2, 192)
        jeven = f85[:, :, :, 0, :]
        jodd = f85[:, :, :, 1, :]
        zcol = jnp.zeros((7, 2, 1, 192), jnp.float32)
        ep_img = jnp.concatenate([zcol, jodd], axis=2)
        op_img = jnp.concatenate([jeven, zcol], axis=2)
        for dy in range(3):
            na = 6 if dy == 0 else 7
            e0r = 0 if dy == 1 else 1
            for dx in range(3):
                img = op_img if dx == 1 else ep_img
                b0 = 1 if dx == 2 else 0
                tv = img[0:na, e0r, b0:b0 + 7, :].reshape(na * 7, 192)
                if dy == 0:
                    tv = jnp.concatenate(
                        [jnp.zeros((7, 192), jnp.float32), tv], axis=0)
                taps[dy * 3 + dx].append(tv)

    acc = jnp.zeros((49 * NT, 192), jnp.float32)
    for tapi in range(9):
        tap_all = jnp.concatenate(taps[tapi], axis=0)
        acc = acc + jnp.dot(tap_all, wc_ref[tapi],
                            preferred_element_type=jnp.float32)
    co_all = acc + b2_ref[0][None, :]

    coT_all = co_all.T
    ys = []
    for i in range(NT):
        yi = jnp.dot(coT_all[:, 49 * i:49 * i + 49], gall_ref[:],
                     preferred_element_type=jnp.float32)
        ys.append(yi * mall_ref[:])
    ycat = jnp.concatenate(ys, axis=1)
    zcat = jnp.dot(rsumT_ref[:], ycat, preferred_element_type=jnp.float32)
    for i in range(NT):
        zall = zcat[:, 1176 * i:1176 * (i + 1)]
        qs = qp_ref[0, i] * (96.0 ** -0.5)
        logits = jnp.zeros((8, 49), jnp.float32)
        for c in range(12):
            logits = logits + zall[:, c * 49:(c + 1) * 49] * qs[:, c:c + 1]
        fv = first_ref[0, i]
        kpr = kp_ref[0, i]
        vpr = vp_ref[0, i]
        logit0 = jnp.sum(qs * kpr, axis=1, keepdims=True) * fv
        m = jnp.maximum(jnp.max(logits, axis=1, keepdims=True), logit0)
        e = jnp.exp(logits - m)
        e0 = jnp.exp(logit0 - m)
        den = jnp.sum(e, axis=1, keepdims=True) + e0
        cols = [jnp.sum(e * zall[:, (12 + c) * 49:(13 + c) * 49], axis=1,
                        keepdims=True) for c in range(12)]
        o8 = (jnp.concatenate(cols, axis=1) + e0 * (vpr * fv)) / den
        out_ref[0, i] = o8


def _proj_body(x_ref, Wo_ref, bo_ref, out_ref):
    out_ref[...] = jnp.dot(x_ref[...], Wo_ref[...],
                           preferred_element_type=jnp.float32) + bo_ref[0][None, :]


@jax.jit
def _run(x, asg, wq_t, bq, wk_t, bk, wv_t, bv, Wq, Wk, Wv, Wc_t, b2, Wo, bo):
    B = x.shape[0]
    prep = pl.pallas_call(
        _prep_body,
        grid=(B,),
        in_specs=[
            pl.BlockSpec((1, T, DIM), lambda b: (b, 0, 0)),
            pl.BlockSpec((9, DIM), lambda b: (0, 0)),
            pl.BlockSpec((1, DIM), lambda b: (0, 0)),
            pl.BlockSpec((9, DIM), lambda b: (0, 0)),
            pl.BlockSpec((1, DIM), lambda b: (0, 0)),
            pl.BlockSpec((9, DIM), lambda b: (0, 0)),
            pl.BlockSpec((1, DIM), lambda b: (0, 0)),
            pl.BlockSpec((DIM, DIM), lambda b: (0, 0)),
            pl.BlockSpec((DIM, DIM), lambda b: (0, 0)),
            pl.BlockSpec((DIM, DIM), lambda b: (0, 0)),
        ],
        out_specs=[
            pl.BlockSpec((1, T, DIM), lambda b: (b, 0, 0)),
            pl.BlockSpec((1, T, DIM), lambda b: (b, 0, 0)),
            pl.BlockSpec((1, T, DIM), lambda b: (b, 0, 0)),
        ],
        out_shape=[jax.ShapeDtypeStruct((B, T, DIM), jnp.float32)] * 3,
    )
    qproj, kproj, vproj = prep(x, wq_t, bq, wk_t, bk, wv_t, bv, Wq, Wk, Wv)

    kh = kproj.reshape(B, T, HEADS, DH).transpose(0, 2, 1, 3)
    vh = vproj.reshape(B, T, HEADS, DH).transpose(0, 2, 1, 3)
    kv2g = jnp.concatenate([kh, vh], axis=-1).reshape(B, G, KV)
    asg2 = asg.reshape(B, G, T)
    asgA = asg2[:, :, 1:].reshape(B, T, 196, 8).transpose(0, 1, 3, 2)
    first_arr = asg[:, :, :, 0].transpose(0, 2, 1).reshape(B, T, HEADS, 1)
    qp4 = qproj.reshape(B, T, HEADS, DH)
    kp4 = kproj.reshape(B, T, HEADS, DH)
    vp4 = vproj.reshape(B, T, HEADS, DH)
    rep = jnp.repeat(jnp.eye(HEADS, dtype=jnp.float32), KV, axis=1)
    pp = jnp.arange(196)
    u0 = (8 * pp) // 196
    rho = (8 * pp) % 196
    oh0 = (jnp.arange(8)[None, :] == u0[:, None]).astype(jnp.float32)
    oh1 = (jnp.arange(8)[None, :] == jnp.minimum(u0 + 1, 7)[:, None]).astype(jnp.float32)
    ohd = oh1 - oh0
    cm8 = ((rho[:, None] + jnp.arange(8)[None, :]) >= 196).astype(jnp.float32)
    cmc = ((rho[:, None] + jnp.arange(2 * DIM)[None, :] // KV) >= 196).astype(jnp.float32)
    sel = ((jnp.arange(2 * DIM)[None, :] % KV) == jnp.arange(KV)[:, None]).astype(jnp.float32)
    cols = jnp.arange(24 * 49)
    mm = 24 * (cols % 49) + cols // 49
    gall = (jnp.arange(49)[:, None] == (mm % 49)[None, :]).astype(jnp.float32)
    mall = ((jnp.arange(2 * DIM)[:, None] % KV) == (mm // 49)[None, :]).astype(jnp.float32)
    rsumT = ((jnp.arange(2 * DIM)[None, :] // KV) == jnp.arange(HEADS)[:, None]).astype(jnp.float32)

    NB = (T + NT - 1) // NT
    out8 = pl.pallas_call(
        _main_body,
        grid=(B, NB),
        in_specs=[
            pl.BlockSpec((1, 8 * NT, T), lambda b, n: (b, n, 0)),
            pl.BlockSpec((1, NT, HEADS, 196), lambda b, n: (b, n, 0, 0)),
            pl.BlockSpec((1, 8 * NT, KV), lambda b, n: (b, n, 0)),
            pl.BlockSpec((1, NT, HEADS, DH), lambda b, n: (b, n, 0, 0)),
            pl.BlockSpec((1, NT, HEADS, DH), lambda b, n: (b, n, 0, 0)),
            pl.BlockSpec((1, NT, HEADS, DH), lambda b, n: (b, n, 0, 0)),
            pl.BlockSpec((1, NT, HEADS, 1), lambda b, n: (b, n, 0, 0)),
            pl.BlockSpec((HEADS, 2 * DIM), lambda b, n: (0, 0)),
            pl.BlockSpec((196, HEADS), lambda b, n: (0, 0)),
            pl.BlockSpec((196, HEADS), lambda b, n: (0, 0)),
            pl.BlockSpec((HEADS, 196), lambda b, n: (0, 0)),
            pl.BlockSpec((HEADS, 196), lambda b, n: (0, 0)),
            pl.BlockSpec((HEADS, 196), lambda b, n: (0, 0)),
            pl.BlockSpec((196, 2 * DIM), lambda b, n: (0, 0)),
            pl.BlockSpec((KV, 2 * DIM), lambda b, n: (0, 0)),
            pl.BlockSpec((9, 2 * DIM, 2 * DIM), lambda b, n: (0, 0, 0)),
            pl.BlockSpec((1, 2 * DIM), lambda b, n: (0, 0)),
            pl.BlockSpec((49, 24 * 49), lambda b, n: (0, 0)),
            pl.BlockSpec((2 * DIM, 24 * 49), lambda b, n: (0, 0)),
            pl.BlockSpec((HEADS, 2 * DIM), lambda b, n: (0, 0)),
        ],
        out_specs=pl.BlockSpec((1, NT, HEADS, DH), lambda b, n: (b, n, 0, 0)),
        out_shape=jax.ShapeDtypeStruct((B, T, HEADS, DH), jnp.float32),
    )(asg2, asgA, kv2g, qp4, kp4, vp4, first_arr, rep, oh0, ohd,
      oh0.T, ohd.T, cm8.T, cmc, sel, Wc_t, b2, gall, mall, rsumT)

    o96 = out8.reshape(B * T, DIM)
    res = pl.pallas_call(
        _proj_body,
        grid=(1,),
        in_specs=[
            pl.BlockSpec((B * T, DIM), lambda i: (0, 0)),
            pl.BlockSpec((DIM, DIM), lambda i: (0, 0)),
            pl.BlockSpec((1, DIM), lambda i: (0, 0)),
        ],
        out_specs=pl.BlockSpec((B * T, DIM), lambda i: (0, 0)),
        out_shape=jax.ShapeDtypeStruct((B * T, DIM), jnp.float32),
    )(o96, Wo, bo)
    return res.reshape(B, T, DIM)


def kernel(x, h, w, attn_score_grad, conv_q_w, bn_q_g, bn_q_b, conv_k_w,
           bn_k_g, bn_k_b, conv_v_w, bn_v_g, bn_v_b, Wq, Wk, Wv, Cw, Cb,
           bn2_g, bn2_b, Wo, bo):
    eps = 1e-5
    sq = bn_q_g / jnp.sqrt(1.0 + eps)
    sk = bn_k_g / jnp.sqrt(1.0 + eps)
    sv = bn_v_g / jnp.sqrt(1.0 + eps)
    wq_t = (conv_q_w[:, 0] * sq[:, None, None]).transpose(1, 2, 0).reshape(9, DIM)
    wk_t = (conv_k_w[:, 0] * sk[:, None, None]).transpose(1, 2, 0).reshape(9, DIM)
    wv_t = (conv_v_w[:, 0] * sv[:, None, None]).transpose(1, 2, 0).reshape(9, DIM)
    s2 = bn2_g / jnp.sqrt(1.0 + eps)
    Wc_t = (Cw * s2[:, None, None, None]).transpose(2, 3, 1, 0).reshape(9, 2 * DIM, 2 * DIM)
    b2 = (Cb * s2 + bn2_b).reshape(1, 2 * DIM)
    return _run(x, attn_score_grad, wq_t, bn_q_b.reshape(1, DIM), wk_t,
                bn_k_b.reshape(1, DIM), wv_t, bn_v_b.reshape(1, DIM),
                Wq, Wk, Wv, Wc_t, b2, Wo, bo.reshape(1, DIM))
